# Initial kernel scaffold; baseline (speedup 1.0000x reference)
import jax
import jax.numpy as jnp
from jax.experimental import pallas as pl


def kernel(user_idx, gender, age, context_idx, user_table, gender_table, age_table, context_table):
    raise NotImplementedError("write your pallas kernel here")


# speedup vs baseline: 16.1743x; 16.1743x over previous
import functools

import jax
import jax.numpy as jnp
from jax import lax
from jax.experimental import pallas as pl
from jax.experimental.pallas import tpu as pltpu
from jax.experimental.pallas import tpu_sc as plsc

B = 16384
HIST = 200
D = 32
OUT = 72

NC = 2
NS = 16
NW = NC * NS
PER_W = B // NW
CB = 16
NCHUNK = PER_W // CB
IDX = 128
NGATHER = CB * HIST // IDX
SCALE = 5.0 / HIST


def _body(uidx_hbm, gend_hbm, age_hbm, cidx_hbm, utbl_hbm, gtbl_hbm,
          atbl_hbm, ctbl_hbm, out_hbm,
          cidx_v, rows_v, uidx_v, gend_v, age_v, urows_v, grows_v, arows_v,
          out_v, sem):
    wid = lax.axis_index("s") * NC + lax.axis_index("c")
    base0 = wid * PER_W

    pltpu.sync_copy(uidx_hbm.at[pl.ds(base0, PER_W)], uidx_v)
    pltpu.sync_copy(gend_hbm.at[pl.ds(base0, PER_W)], gend_v)
    pltpu.sync_copy(age_hbm.at[pl.ds(base0, PER_W)], age_v)

    def chunk_body(c, carry):
        base = base0 + c * CB
        pltpu.sync_copy(cidx_hbm.at[pl.ds(base * HIST, CB * HIST)], cidx_v)
        copies = [
            pltpu.async_copy(
                ctbl_hbm.at[cidx_v.at[pl.ds(j * IDX, IDX)]],
                rows_v.at[pl.ds(j * IDX, IDX)],
                sem,
            )
            for j in range(NGATHER)
        ]
        copies.append(pltpu.async_copy(
            utbl_hbm.at[uidx_v.at[pl.ds(c * CB, CB)]], urows_v, sem))
        copies.append(pltpu.async_copy(
            gtbl_hbm.at[gend_v.at[pl.ds(c * CB, CB)]], grows_v, sem))
        copies.append(pltpu.async_copy(
            atbl_hbm.at[age_v.at[pl.ds(c * CB, CB)]], arows_v, sem))
        for cp in copies:
            cp.wait()

        def pool(b, carry2):
            def red(h, accs):
                a0, a1, b0, b1 = accs
                r = b * HIST + h * 4
                a0 = a0 + rows_v[r, pl.ds(0, 16)]
                a1 = a1 + rows_v[r, pl.ds(16, 16)]
                b0 = b0 + rows_v[r + 1, pl.ds(0, 16)]
                b1 = b1 + rows_v[r + 1, pl.ds(16, 16)]
                a0 = a0 + rows_v[r + 2, pl.ds(0, 16)]
                a1 = a1 + rows_v[r + 2, pl.ds(16, 16)]
                b0 = b0 + rows_v[r + 3, pl.ds(0, 16)]
                b1 = b1 + rows_v[r + 3, pl.ds(16, 16)]
                return a0, a1, b0, b1

            zero = jnp.zeros((16,), jnp.float32)
            a0, a1, b0, b1 = lax.fori_loop(
                0, HIST // 4, red, (zero, zero, zero, zero))
            out_v[pl.ds(b * OUT, 16)] = urows_v[b, pl.ds(0, 16)]
            out_v[pl.ds(b * OUT + 16, 16)] = urows_v[b, pl.ds(16, 16)]
            out_v[pl.ds(b * OUT + 32, 16)] = grows_v[b, pl.ds(0, 16)]
            out_v[pl.ds(b * OUT + 36, 16)] = arows_v[b, pl.ds(0, 16)]
            out_v[pl.ds(b * OUT + 40, 16)] = (a0 + b0) * SCALE
            out_v[pl.ds(b * OUT + 56, 16)] = (a1 + b1) * SCALE
            return carry2

        lax.fori_loop(0, CB, pool, 0)

        pltpu.sync_copy(out_v, out_hbm.at[pl.ds(base * OUT, CB * OUT)])
        return carry

    lax.fori_loop(0, NCHUNK, chunk_body, 0)


@functools.lru_cache(maxsize=None)
def _build(interpret: bool = False):
    return functools.partial(
        pl.kernel,
        out_type=jax.ShapeDtypeStruct((B * OUT,), jnp.float32),
        mesh=plsc.VectorSubcoreMesh(core_axis_name="c", subcore_axis_name="s",
                                    num_cores=NC, num_subcores=NS),
        scratch_types=[
            pltpu.VMEM((CB * HIST,), jnp.int32),
            pltpu.VMEM((CB * HIST, D), jnp.float32),
            pltpu.VMEM((PER_W,), jnp.int32),
            pltpu.VMEM((PER_W,), jnp.int32),
            pltpu.VMEM((PER_W,), jnp.int32),
            pltpu.VMEM((CB, D), jnp.float32),
            pltpu.VMEM((CB, 16), jnp.float32),
            pltpu.VMEM((CB, 16), jnp.float32),
            pltpu.VMEM((CB * OUT,), jnp.float32),
            pltpu.SemaphoreType.DMA,
        ],
        compiler_params=pltpu.CompilerParams(use_tc_tiling_on_sc=False),
        interpret=interpret,
    )(lambda *refs: _body(*refs))


def kernel(user_idx, gender, age, context_idx, user_table, gender_table,
           age_table, context_table):
    gtbl = jnp.zeros((8, 16), jnp.float32).at[:3, :4].set(gender_table)
    atbl = jnp.zeros((104, 16), jnp.float32).at[:100, :4].set(age_table)
    out = _build()(
        user_idx.astype(jnp.int32),
        gender.astype(jnp.int32),
        age.astype(jnp.int32),
        context_idx.reshape(-1).astype(jnp.int32),
        user_table,
        gtbl,
        atbl,
        context_table,
    )
    return out.reshape(B, OUT)


# speedup vs baseline: 16.1998x; 1.0016x over previous
import functools

import jax
import jax.numpy as jnp
from jax import lax
from jax.experimental import pallas as pl
from jax.experimental.pallas import tpu as pltpu
from jax.experimental.pallas import tpu_sc as plsc

B = 16384
HIST = 200
D = 32
OUT = 72

NC = 2
NS = 16
NW = NC * NS
PER_W = B // NW
CB = 16
NCHUNK = PER_W // CB
IDX = 3200
NGATHER = CB * HIST // IDX
SCALE = 5.0 / HIST


def _body(uidx_hbm, gend_hbm, age_hbm, cidx_hbm, utbl_hbm, gtbl_hbm,
          atbl_hbm, ctbl_hbm, out_hbm,
          cidx_v, rows_v, uidx_v, gend_v, age_v, urows_v, grows_v, arows_v,
          out_v, sem):
    wid = lax.axis_index("s") * NC + lax.axis_index("c")
    base0 = wid * PER_W

    pltpu.sync_copy(uidx_hbm.at[pl.ds(base0, PER_W)], uidx_v)
    pltpu.sync_copy(gend_hbm.at[pl.ds(base0, PER_W)], gend_v)
    pltpu.sync_copy(age_hbm.at[pl.ds(base0, PER_W)], age_v)

    def chunk_body(c, carry):
        base = base0 + c * CB
        pltpu.sync_copy(cidx_hbm.at[pl.ds(base * HIST, CB * HIST)], cidx_v)
        copies = [
            pltpu.async_copy(
                ctbl_hbm.at[cidx_v.at[pl.ds(j * IDX, IDX)]],
                rows_v.at[pl.ds(j * IDX, IDX)],
                sem,
            )
            for j in range(NGATHER)
        ]
        copies.append(pltpu.async_copy(
            utbl_hbm.at[uidx_v.at[pl.ds(c * CB, CB)]], urows_v, sem))
        copies.append(pltpu.async_copy(
            gtbl_hbm.at[gend_v.at[pl.ds(c * CB, CB)]], grows_v, sem))
        copies.append(pltpu.async_copy(
            atbl_hbm.at[age_v.at[pl.ds(c * CB, CB)]], arows_v, sem))
        for cp in copies:
            cp.wait()

        def pool(b, carry2):
            def red(h, accs):
                a0, a1, b0, b1 = accs
                r = b * HIST + h * 4
                a0 = a0 + rows_v[r, pl.ds(0, 16)]
                a1 = a1 + rows_v[r, pl.ds(16, 16)]
                b0 = b0 + rows_v[r + 1, pl.ds(0, 16)]
                b1 = b1 + rows_v[r + 1, pl.ds(16, 16)]
                a0 = a0 + rows_v[r + 2, pl.ds(0, 16)]
                a1 = a1 + rows_v[r + 2, pl.ds(16, 16)]
                b0 = b0 + rows_v[r + 3, pl.ds(0, 16)]
                b1 = b1 + rows_v[r + 3, pl.ds(16, 16)]
                return a0, a1, b0, b1

            zero = jnp.zeros((16,), jnp.float32)
            a0, a1, b0, b1 = lax.fori_loop(
                0, HIST // 4, red, (zero, zero, zero, zero))
            out_v[pl.ds(b * OUT, 16)] = urows_v[b, pl.ds(0, 16)]
            out_v[pl.ds(b * OUT + 16, 16)] = urows_v[b, pl.ds(16, 16)]
            out_v[pl.ds(b * OUT + 32, 16)] = grows_v[b, pl.ds(0, 16)]
            out_v[pl.ds(b * OUT + 36, 16)] = arows_v[b, pl.ds(0, 16)]
            out_v[pl.ds(b * OUT + 40, 16)] = (a0 + b0) * SCALE
            out_v[pl.ds(b * OUT + 56, 16)] = (a1 + b1) * SCALE
            return carry2

        lax.fori_loop(0, CB, pool, 0)

        pltpu.sync_copy(out_v, out_hbm.at[pl.ds(base * OUT, CB * OUT)])
        return carry

    lax.fori_loop(0, NCHUNK, chunk_body, 0)


@functools.lru_cache(maxsize=None)
def _build(interpret: bool = False):
    return functools.partial(
        pl.kernel,
        out_type=jax.ShapeDtypeStruct((B * OUT,), jnp.float32),
        mesh=plsc.VectorSubcoreMesh(core_axis_name="c", subcore_axis_name="s",
                                    num_cores=NC, num_subcores=NS),
        scratch_types=[
            pltpu.VMEM((CB * HIST,), jnp.int32),
            pltpu.VMEM((CB * HIST, D), jnp.float32),
            pltpu.VMEM((PER_W,), jnp.int32),
            pltpu.VMEM((PER_W,), jnp.int32),
            pltpu.VMEM((PER_W,), jnp.int32),
            pltpu.VMEM((CB, D), jnp.float32),
            pltpu.VMEM((CB, 16), jnp.float32),
            pltpu.VMEM((CB, 16), jnp.float32),
            pltpu.VMEM((CB * OUT,), jnp.float32),
            pltpu.SemaphoreType.DMA,
        ],
        compiler_params=pltpu.CompilerParams(use_tc_tiling_on_sc=False),
        interpret=interpret,
    )(lambda *refs: _body(*refs))


def kernel(user_idx, gender, age, context_idx, user_table, gender_table,
           age_table, context_table):
    gtbl = jnp.zeros((8, 16), jnp.float32).at[:3, :4].set(gender_table)
    atbl = jnp.zeros((104, 16), jnp.float32).at[:100, :4].set(age_table)
    out = _build()(
        user_idx.astype(jnp.int32),
        gender.astype(jnp.int32),
        age.astype(jnp.int32),
        context_idx.reshape(-1).astype(jnp.int32),
        user_table,
        gtbl,
        atbl,
        context_table,
    )
    return out.reshape(B, OUT)


# speedup vs baseline: 17.1763x; 1.0603x over previous
import functools

import jax
import jax.numpy as jnp
from jax import lax
from jax.experimental import pallas as pl
from jax.experimental.pallas import tpu as pltpu
from jax.experimental.pallas import tpu_sc as plsc

B = 16384
HIST = 200
D = 32
OUT = 72

NC = 2
NS = 16
NW = NC * NS
PER_W = B // NW
CB = 8
NCHUNK = PER_W // CB
SCALE = 5.0 / HIST


def _body(uidx_hbm, gend_hbm, age_hbm, cidx_hbm, utbl_hbm, gtbl_hbm,
          atbl_hbm, ctbl_hbm, out_hbm,
          cidx_v, rows_v, uidx_v, gend_v, age_v, urows_v, grows_v, arows_v,
          out_v, semg0, semg1, semw0, semw1):
    semg = (semg0, semg1)
    semw = (semw0, semw1)
    wid = lax.axis_index("s") * NC + lax.axis_index("c")
    base0 = wid * PER_W

    pltpu.sync_copy(uidx_hbm.at[pl.ds(base0, PER_W)], uidx_v)
    pltpu.sync_copy(gend_hbm.at[pl.ds(base0, PER_W)], gend_v)
    pltpu.sync_copy(age_hbm.at[pl.ds(base0, PER_W)], age_v)

    def issue(c, ph):
        base = base0 + c * CB
        pltpu.sync_copy(cidx_hbm.at[pl.ds(base * HIST, CB * HIST)],
                        cidx_v.at[ph])
        pltpu.async_copy(ctbl_hbm.at[cidx_v.at[ph]], rows_v.at[ph], semg[ph])
        pltpu.async_copy(utbl_hbm.at[uidx_v.at[pl.ds(c * CB, CB)]],
                         urows_v.at[ph], semg[ph])
        pltpu.async_copy(gtbl_hbm.at[gend_v.at[pl.ds(c * CB, CB)]],
                         grows_v.at[ph], semg[ph])
        pltpu.async_copy(atbl_hbm.at[age_v.at[pl.ds(c * CB, CB)]],
                         arows_v.at[ph], semg[ph])

    def wait_gathers(ph):
        pltpu.make_async_copy(ctbl_hbm.at[pl.ds(0, CB * HIST)],
                              rows_v.at[ph], semg[ph]).wait()
        pltpu.make_async_copy(utbl_hbm.at[pl.ds(0, CB)],
                              urows_v.at[ph], semg[ph]).wait()
        pltpu.make_async_copy(gtbl_hbm.at[pl.ds(0, CB)],
                              grows_v.at[ph], semg[ph]).wait()
        pltpu.make_async_copy(atbl_hbm.at[pl.ds(0, CB)],
                              arows_v.at[ph], semg[ph]).wait()

    def drain_out(ph):
        pltpu.make_async_copy(out_v.at[ph],
                              out_hbm.at[pl.ds(0, CB * OUT)], semw[ph]).wait()

    def compute(c, ph):
        rows = rows_v.at[ph]
        out = out_v.at[ph]

        def pool(b, carry2):
            def red(h, accs):
                a0, a1, b0, b1 = accs
                r = b * HIST + h * 4
                a0 = a0 + rows[r, pl.ds(0, 16)]
                a1 = a1 + rows[r, pl.ds(16, 16)]
                b0 = b0 + rows[r + 1, pl.ds(0, 16)]
                b1 = b1 + rows[r + 1, pl.ds(16, 16)]
                a0 = a0 + rows[r + 2, pl.ds(0, 16)]
                a1 = a1 + rows[r + 2, pl.ds(16, 16)]
                b0 = b0 + rows[r + 3, pl.ds(0, 16)]
                b1 = b1 + rows[r + 3, pl.ds(16, 16)]
                return a0, a1, b0, b1

            zero = jnp.zeros((16,), jnp.float32)
            a0, a1, b0, b1 = lax.fori_loop(
                0, HIST // 4, red, (zero, zero, zero, zero))
            out[pl.ds(b * OUT, 16)] = urows_v[ph, b, pl.ds(0, 16)]
            out[pl.ds(b * OUT + 16, 16)] = urows_v[ph, b, pl.ds(16, 16)]
            out[pl.ds(b * OUT + 32, 16)] = grows_v[ph, b, pl.ds(0, 16)]
            out[pl.ds(b * OUT + 36, 16)] = arows_v[ph, b, pl.ds(0, 16)]
            out[pl.ds(b * OUT + 40, 16)] = (a0 + b0) * SCALE
            out[pl.ds(b * OUT + 56, 16)] = (a1 + b1) * SCALE
            return carry2

        lax.fori_loop(0, CB, pool, 0)
        base = base0 + c * CB
        pltpu.async_copy(out_v.at[ph], out_hbm.at[pl.ds(base * OUT, CB * OUT)],
                         semw[ph])

    issue(0, 0)

    def pair_body(p, carry):
        for ph in range(2):
            c = p * 2 + ph

            @pl.when(c + 1 < NCHUNK)
            def _():
                issue(c + 1, 1 - ph)

            wait_gathers(ph)

            @pl.when(c >= 2)
            def _():
                drain_out(ph)

            compute(c, ph)
        return carry

    lax.fori_loop(0, NCHUNK // 2, pair_body, 0)
    drain_out(0)
    drain_out(1)


@functools.lru_cache(maxsize=None)
def _build(interpret: bool = False):
    return functools.partial(
        pl.kernel,
        out_type=jax.ShapeDtypeStruct((B * OUT,), jnp.float32),
        mesh=plsc.VectorSubcoreMesh(core_axis_name="c", subcore_axis_name="s",
                                    num_cores=NC, num_subcores=NS),
        scratch_types=[
            pltpu.VMEM((2, CB * HIST,), jnp.int32),
            pltpu.VMEM((2, CB * HIST, D), jnp.float32),
            pltpu.VMEM((PER_W,), jnp.int32),
            pltpu.VMEM((PER_W,), jnp.int32),
            pltpu.VMEM((PER_W,), jnp.int32),
            pltpu.VMEM((2, CB, D), jnp.float32),
            pltpu.VMEM((2, CB, 16), jnp.float32),
            pltpu.VMEM((2, CB, 16), jnp.float32),
            pltpu.VMEM((2, CB * OUT), jnp.float32),
            pltpu.SemaphoreType.DMA,
            pltpu.SemaphoreType.DMA,
            pltpu.SemaphoreType.DMA,
            pltpu.SemaphoreType.DMA,
        ],
        compiler_params=pltpu.CompilerParams(use_tc_tiling_on_sc=False),
        interpret=interpret,
    )(lambda *refs: _body(*refs))


def kernel(user_idx, gender, age, context_idx, user_table, gender_table,
           age_table, context_table):
    gtbl = jnp.zeros((8, 16), jnp.float32).at[:3, :4].set(gender_table)
    atbl = jnp.zeros((104, 16), jnp.float32).at[:100, :4].set(age_table)
    out = _build()(
        user_idx.astype(jnp.int32),
        gender.astype(jnp.int32),
        age.astype(jnp.int32),
        context_idx.reshape(-1).astype(jnp.int32),
        user_table,
        gtbl,
        atbl,
        context_table,
    )
    return out.reshape(B, OUT)
